# baseline (speedup 1.0000x reference)
import functools

import jax
import jax.numpy as jnp
from jax.experimental import pallas as pl
from jax.experimental.pallas import tpu as pltpu

_TOP_K = 8


def _pool_kernel(x_any, out_ref, buf0, buf1, sem0, sem1, *, n_seg, hw, s_chunk):
    xs = x_any
    bufs = (buf0, buf1)
    sems = (sem0, sem1)
    n_chunks = n_seg // s_chunk

    def _copy(i):
        return pltpu.make_async_copy(
            xs.at[pl.ds(i * s_chunk, s_chunk), :], bufs[i % 2], sems[i % 2]
        )

    _copy(0).start()
    for i in range(n_chunks):
        if i + 1 < n_chunks:
            _copy(i + 1).start()
        _copy(i).wait()
        xt = bufs[i % 2][...]
        red = jnp.max(xt, axis=1) + jnp.sum(xt, axis=1) * (1.0 / hw)
        out_ref[pl.ds(i * (s_chunk // 32), s_chunk // 32), :] = red.reshape(
            s_chunk // 32, 32
        )


def _head_kernel(p_ref, w0_ref, b0_ref, w1_ref, b1_ref, out_ref):
    pooled = p_ref[...]
    dn = (((1,), (1,)), ((), ()))
    h = jax.lax.dot_general(pooled, w1_ref[...], dn,
                            preferred_element_type=jnp.float32) + b1_ref[...]
    h = jnp.where(h >= 0, h, 0.2 * h)

    z = jax.lax.dot_general(pooled, w0_ref[...], dn,
                            preferred_element_type=jnp.float32) + b0_ref[...]
    noise = jnp.maximum(z, 0.0) + jnp.log1p(jnp.exp(-jnp.abs(z)))

    e = noise.shape[1]
    nmean = jnp.mean(noise, axis=1, keepdims=True)
    d = noise - nmean
    var = jnp.sum(d * d, axis=1, keepdims=True) * (1.0 / (e - 1))
    std = jnp.sqrt(var)
    std = jnp.where(std == 0, 1.0, std)
    scores = h + d / std

    iota = jax.lax.broadcasted_iota(jnp.int32, scores.shape, 1)
    work = scores
    mask = jnp.zeros_like(scores, dtype=jnp.bool_)
    for _ in range(_TOP_K):
        m = jnp.max(work, axis=1, keepdims=True)
        first = jnp.min(jnp.where(work == m, iota, e), axis=1, keepdims=True)
        sel = iota == first
        mask = jnp.logical_or(mask, sel)
        work = jnp.where(sel, -1e30, work)

    h_masked = jnp.where(mask, h, -1e9)
    hm = jnp.max(h_masked, axis=1, keepdims=True)
    ex = jnp.exp(h_masked - hm)
    out_ref[...] = ex / jnp.sum(ex, axis=1, keepdims=True)


@functools.partial(jax.jit, static_argnames=("interpret",))
def kernel(x, W0, b0, W1, b1, interpret=False):
    B, C, H, W = x.shape
    E = W0.shape[0]
    hw = H * W
    n_seg = B * C
    s_chunk = 8192

    x2 = x.reshape(n_seg, hw)
    pooled32 = pl.pallas_call(
        functools.partial(_pool_kernel, n_seg=n_seg, hw=hw, s_chunk=s_chunk),
        in_specs=[pl.BlockSpec(memory_space=pl.ANY)],
        out_specs=pl.BlockSpec(memory_space=pltpu.VMEM),
        out_shape=jax.ShapeDtypeStruct((n_seg // 32, 32), jnp.float32),
        scratch_shapes=[
            pltpu.VMEM((s_chunk, hw), jnp.float32),
            pltpu.VMEM((s_chunk, hw), jnp.float32),
            pltpu.SemaphoreType.DMA,
            pltpu.SemaphoreType.DMA,
        ],
        interpret=interpret,
    )(x2)
    pooled = pooled32.reshape(B, C)

    out = pl.pallas_call(
        _head_kernel,
        out_shape=jax.ShapeDtypeStruct((B, E), jnp.float32),
        interpret=interpret,
    )(pooled, W0, b0.reshape(1, E), W1, b1.reshape(1, E))
    return out
